# Initial kernel scaffold; baseline (speedup 1.0000x reference)
import jax
import jax.numpy as jnp
from jax.experimental import pallas as pl


def kernel(x, edge_index, edge_attr, batch, params):
    raise NotImplementedError("write your pallas kernel here")


# speedup vs baseline: 13.3868x; 13.3868x over previous
import functools

import jax
import jax.numpy as jnp
import numpy as np
from jax import lax
from jax.experimental import pallas as pl
from jax.experimental.pallas import tpu as pltpu
from jax.experimental.pallas import tpu_sc as plsc

_T = 6
_NG = 64
_DEG_HIST = np.concatenate([np.zeros(16), np.array([10000.0])])
_BINS = np.arange(len(_DEG_HIST), dtype=np.float64)
_AVG_LOG = float((np.log(_BINS + 1.0) * _DEG_HIST).sum() / _DEG_HIST.sum())

_CH = 24
_NW = 32
_ROW_BLK = 400
_EC_BLK = 1000


def _ab_body(h_ref, wd_ref, ws_ref, pb_ref, a_ref, b_ref):
    h = h_ref[...]
    a_ref[...] = jnp.dot(h, wd_ref[...], preferred_element_type=jnp.float32) + pb_ref[...]
    b_ref[...] = jnp.dot(h, ws_ref[...], preferred_element_type=jnp.float32)


def _compute_ab(h, wd, ws, pb, kpad):
    n = h.shape[0]
    fin = h.shape[1]
    grid = (n // _ROW_BLK,)
    return pl.pallas_call(
        _ab_body,
        grid=grid,
        in_specs=[
            pl.BlockSpec((_ROW_BLK, fin), lambda i: (i, 0)),
            pl.BlockSpec((fin, kpad), lambda i: (0, 0)),
            pl.BlockSpec((fin, kpad), lambda i: (0, 0)),
            pl.BlockSpec((1, kpad), lambda i: (0, 0)),
        ],
        out_specs=[
            pl.BlockSpec((_ROW_BLK, kpad), lambda i: (i, 0)),
            pl.BlockSpec((_ROW_BLK, kpad), lambda i: (i, 0)),
        ],
        out_shape=[
            jax.ShapeDtypeStruct((n, kpad), jnp.float32),
            jax.ShapeDtypeStruct((n, kpad), jnp.float32),
        ],
    )(h, wd, ws, pb)


def _c_body(ea_ref, wc_ref, cb_ref, c_ref):
    ea = ea_ref[...]
    c_ref[...] = jnp.dot(ea, wc_ref[...], preferred_element_type=jnp.float32) + cb_ref[...]


def _compute_c(ea_s, wc, cb, kpad):
    epad = ea_s.shape[0]
    grid = (epad // _EC_BLK,)
    return pl.pallas_call(
        _c_body,
        grid=grid,
        in_specs=[
            pl.BlockSpec((_EC_BLK, 4), lambda i: (i, 0)),
            pl.BlockSpec((4, kpad), lambda i: (0, 0)),
            pl.BlockSpec((1, kpad), lambda i: (0, 0)),
        ],
        out_specs=pl.BlockSpec((_EC_BLK, kpad), lambda i: (i, 0)),
        out_shape=jax.ShapeDtypeStruct((epad, kpad), jnp.float32),
    )(ea_s, wc, cb)


def _sc_seg_stats(b_rows, c_rows, src_s, dst_s, tes, n_nodes, kpad):
    k16 = kpad // 16
    big = jnp.float32(3.0e38)
    mesh = plsc.VectorSubcoreMesh(core_axis_name="c", subcore_axis_name="s",
                                  num_cores=2, num_subcores=16)
    out_sd = jax.ShapeDtypeStruct((n_nodes, kpad), jnp.float32)

    @functools.partial(
        pl.kernel,
        out_type=(out_sd, out_sd, out_sd, out_sd),
        mesh=mesh,
        compiler_params=pltpu.CompilerParams(needs_layout_passes=False),
        scratch_types=[
            pltpu.VMEM((48,), jnp.int32),
            pltpu.VMEM((_CH,), jnp.int32),
            pltpu.VMEM((_CH,), jnp.int32),
            pltpu.VMEM((2, _CH + 16), jnp.int32),
            pltpu.VMEM((2, _CH, kpad), jnp.float32),
            pltpu.VMEM((2, _CH, kpad), jnp.float32),
            pltpu.VMEM((4, kpad), jnp.float32),
            pltpu.SemaphoreType.DMA,
            pltpu.SemaphoreType.DMA,
            pltpu.SemaphoreType.DMA,
        ],
    )
    def kern(b_hbm, c_hbm, src_hbm, dst_hbm, tes_hbm,
             su_hbm, sq_hbm, mn_hbm, mx_hbm,
             tes_v, idx0_v, idx1_v, dst_v, rows_v, c_v, acc_v,
             semf0, semf1, semo):
        wid = lax.axis_index("s") * 2 + lax.axis_index("c")
        pltpu.sync_copy(tes_hbm, tes_v)
        tv = tes_v[pl.ds(wid, 16)]
        e0 = tv[0]
        e1 = tv[1]
        g0 = e0 // _CH
        ng = jnp.maximum((e1 + _CH - 1) // _CH - g0, 0)
        semf = (semf0, semf1)
        idxs = (idx0_v, idx1_v)
        out_refs = (su_hbm, sq_hbm, mn_hbm, mx_hbm)

        def fetch(buf, g):
            pltpu.sync_copy(src_hbm.at[pl.ds(g * _CH, _CH)], idxs[buf])
            pltpu.sync_copy(dst_hbm.at[pl.ds(g * _CH, _CH)],
                            dst_v.at[buf, pl.ds(0, _CH)])
            pltpu.async_copy(b_hbm.at[idxs[buf]], rows_v.at[buf], semf[buf])
            pltpu.async_copy(c_hbm.at[pl.ds(g * _CH, _CH)], c_v.at[buf], semf[buf])

        def wait_fetch(buf):
            pltpu.make_async_copy(b_hbm.at[idxs[buf]], rows_v.at[buf], semf[buf]).wait()
            pltpu.make_async_copy(c_hbm.at[pl.ds(0, _CH)], c_v.at[buf], semf[buf]).wait()

        def flush(node):
            cps = [pltpu.async_copy(acc_v.at[j], out_refs[j].at[node], semo)
                   for j in range(4)]
            for cp in cps:
                cp.wait()

        def init_acc():
            def ibody(k, _):
                ks = pl.ds(k * 16, 16)
                z = jnp.zeros((16,), jnp.float32)
                acc_v[0, ks] = z
                acc_v[1, ks] = z
                acc_v[2, ks] = jnp.full((16,), big, jnp.float32)
                acc_v[3, ks] = jnp.full((16,), -big, jnp.float32)
                return 0
            lax.fori_loop(0, k16, ibody, 0)

        def process_chunk(buf, g, cur):
            lo = jnp.maximum(e0 - g * _CH, 0)
            hi = jnp.minimum(e1 - g * _CH, _CH)
            lane = lax.broadcasted_iota(jnp.int32, (16,), 0)

            def run_body(_, st):
                i, cur = st
                valid = i < hi
                isafe = jnp.minimum(i, _CH - 1)
                w = dst_v[buf, pl.ds(isafe, 16)]
                n = w[0]
                is_new = valid & (n != cur)

                @pl.when(is_new & (cur >= 0))
                def _():
                    flush(cur)

                @pl.when(is_new)
                def _():
                    init_acc()

                cap = jnp.minimum(hi - isafe, 15)
                stop = (w != n) | (lane >= cap)
                rl = jnp.min(jnp.where(stop, lane, 16))
                j = isafe + jnp.maximum(rl, 1)

                @pl.when(valid)
                def _():
                    def kbody(k, _):
                        ks = pl.ds(k * 16, 16)
                        s1 = acc_v[0, ks]
                        s2 = acc_v[1, ks]
                        mn = acc_v[2, ks]
                        mx = acc_v[3, ks]

                        def ebody(e, c4):
                            s1, s2, mn, mx = c4
                            u = rows_v[buf, e, ks] + c_v[buf, e, ks]
                            return (s1 + u, s2 + u * u,
                                    jnp.minimum(mn, u), jnp.maximum(mx, u))

                        s1, s2, mn, mx = lax.fori_loop(isafe, j, ebody,
                                                       (s1, s2, mn, mx))
                        acc_v[0, ks] = s1
                        acc_v[1, ks] = s2
                        acc_v[2, ks] = mn
                        acc_v[3, ks] = mx
                        return 0

                    lax.fori_loop(0, k16, kbody, 0)

                i_next = jnp.where(valid, j, i)
                cur_next = jnp.where(valid, n, cur)
                return (i_next, cur_next)

            _, cur = lax.fori_loop(0, _CH, run_body, (lo, cur))
            return cur

        @pl.when(ng > 0)
        def _():
            fetch(0, g0)

        @pl.when(ng > 1)
        def _():
            fetch(1, g0 + 1)

        def pair_body(p, cur):
            for half in (0, 1):
                gi = 2 * p + half
                g = g0 + gi

                @pl.when(gi < ng)
                def _():
                    wait_fetch(half)

                cur = process_chunk(half, g, cur)

                @pl.when(gi + 2 < ng)
                def _():
                    fetch(half, g + 2)
            return cur

        cur = lax.fori_loop(0, (ng + 1) // 2, pair_body, jnp.int32(-1))

        @pl.when(cur >= 0)
        def _():
            flush(cur)

    return kern(b_rows, c_rows, src_s, dst_s, tes)


def _post_body(su_ref, sq_ref, mn_ref, mx_ref, a_ref, h_ref, deg_ref,
               wagg_ref, wamp_ref, watt_ref, wx_ref, pb_ref, lw_ref, lb_ref,
               out_ref, *, do_relu):
    deg = deg_ref[...]
    a = a_ref[...]
    su = su_ref[...]
    has = deg > 0.0
    cnt = jnp.maximum(deg, 1.0)
    mean = jnp.where(has, (deg * a + su) / cnt, 0.0)
    s2 = deg * a * a + 2.0 * a * su + sq_ref[...]
    var = jnp.where(has, s2 / cnt - mean * mean, 0.0)
    std = jnp.sqrt(jnp.maximum(var, 0.0) + 1e-5)
    mn = jnp.where(has, a + mn_ref[...], 0.0)
    mx = jnp.where(has, a + mx_ref[...], 0.0)
    g = jnp.concatenate([mean, mn, mx, std], axis=1)
    dc = jnp.maximum(deg, 1.0)
    ldc = jnp.log(dc + 1.0)
    sa = ldc * jnp.float32(1.0 / _AVG_LOG)
    st = jnp.float32(_AVG_LOG) / ldc
    p1 = jnp.dot(g, wagg_ref[...], preferred_element_type=jnp.float32)
    p2 = jnp.dot(g, wamp_ref[...], preferred_element_type=jnp.float32)
    p3 = jnp.dot(g, watt_ref[...], preferred_element_type=jnp.float32)
    px = jnp.dot(h_ref[...], wx_ref[...], preferred_element_type=jnp.float32)
    p = px + p1 + sa * p2 + st * p3 + pb_ref[...]
    o = jnp.dot(p, lw_ref[...], preferred_element_type=jnp.float32) + lb_ref[...]
    if do_relu:
        o = jnp.maximum(o, 0.0)
    out_ref[...] = o


def _compute_post(su, sq, mn, mx, a, h, deg, wagg, wamp, watt, wx, pb, lw, lb,
                  do_relu):
    n, kpad = su.shape
    fin = h.shape[1]
    emb = lw.shape[0]
    grid = (n // _ROW_BLK,)
    row = lambda i: (i, 0)
    cst = lambda i: (0, 0)
    return pl.pallas_call(
        functools.partial(_post_body, do_relu=do_relu),
        grid=grid,
        in_specs=[
            pl.BlockSpec((_ROW_BLK, kpad), row),
            pl.BlockSpec((_ROW_BLK, kpad), row),
            pl.BlockSpec((_ROW_BLK, kpad), row),
            pl.BlockSpec((_ROW_BLK, kpad), row),
            pl.BlockSpec((_ROW_BLK, kpad), row),
            pl.BlockSpec((_ROW_BLK, fin), row),
            pl.BlockSpec((_ROW_BLK, 1), row),
            pl.BlockSpec((4 * kpad, emb), cst),
            pl.BlockSpec((4 * kpad, emb), cst),
            pl.BlockSpec((4 * kpad, emb), cst),
            pl.BlockSpec((fin, emb), cst),
            pl.BlockSpec((1, emb), cst),
            pl.BlockSpec((emb, emb), cst),
            pl.BlockSpec((1, emb), cst),
        ],
        out_specs=pl.BlockSpec((_ROW_BLK, emb), row),
        out_shape=jax.ShapeDtypeStruct((n, emb), jnp.float32),
    )(su, sq, mn, mx, a, h, deg, wagg, wamp, watt, wx, pb, lw, lb)


def _pool_body(h_ref, batch_ref, w1_ref, b1_ref, w2_ref, b2_ref,
               w3_ref, b3_ref, out_ref):
    n = h_ref.shape[0]
    bvec = batch_ref[...]
    gid = lax.broadcasted_iota(jnp.int32, (_NG, n), 0)
    oh = jnp.where(gid == bvec, 1.0, 0.0).astype(jnp.float32)
    g = jnp.dot(oh, h_ref[...], preferred_element_type=jnp.float32)
    g = jnp.maximum(jnp.dot(g, w1_ref[...], preferred_element_type=jnp.float32)
                    + b1_ref[...], 0.0)
    g = jnp.maximum(jnp.dot(g, w2_ref[...], preferred_element_type=jnp.float32)
                    + b2_ref[...], 0.0)
    g = jnp.maximum(jnp.dot(g, w3_ref[...], preferred_element_type=jnp.float32)
                    + b3_ref[...], 0.0)
    out_ref[...] = g


def _compute_pool(h, batch, w1, b1, w2, b2, w3, b3):
    n, emb = h.shape
    d1 = w1.shape[1]
    return pl.pallas_call(
        _pool_body,
        in_specs=[
            pl.BlockSpec((n, emb), lambda: (0, 0)),
            pl.BlockSpec((1, n), lambda: (0, 0)),
            pl.BlockSpec((emb, d1), lambda: (0, 0)),
            pl.BlockSpec((1, d1), lambda: (0, 0)),
            pl.BlockSpec((d1, emb), lambda: (0, 0)),
            pl.BlockSpec((1, emb), lambda: (0, 0)),
            pl.BlockSpec((emb, 8), lambda: (0, 0)),
            pl.BlockSpec((1, 8), lambda: (0, 0)),
        ],
        out_specs=pl.BlockSpec((_NG, 8), lambda: (0, 0)),
        out_shape=jax.ShapeDtypeStruct((_NG, 8), jnp.float32),
    )(h, batch, w1, b1, w2, b2, w3, b3)


def _layer_weights(p, fin, kpad):
    k = _T * fin
    preW = p["preW"]
    wd = jnp.transpose(preW[:, :fin], (1, 0, 2)).reshape(fin, k)
    ws = jnp.transpose(preW[:, fin:2 * fin], (1, 0, 2)).reshape(fin, k)
    we2 = preW[:, 2 * fin:]
    wc = jnp.einsum("df,tfg->dtg", p["We"], we2).reshape(4, k)
    cb = (jnp.einsum("f,tfg->tg", p["be"], we2)).reshape(1, k)
    pb = p["preb"].reshape(1, k)
    if kpad != k:
        pad = kpad - k
        wd = jnp.pad(wd, ((0, 0), (0, pad)))
        ws = jnp.pad(ws, ((0, 0), (0, pad)))
        wc = jnp.pad(wc, ((0, 0), (0, pad)))
        cb = jnp.pad(cb, ((0, 0), (0, pad)))
        pb = jnp.pad(pb, ((0, 0), (0, pad)))
    postW = p["postW"]
    fout = postW.shape[2]
    emb = _T * fout

    def bd(wpart):
        w = jnp.zeros((4 * kpad, emb), jnp.float32)
        for s in range(4):
            for t in range(_T):
                w = w.at[s * kpad + t * fin:s * kpad + (t + 1) * fin,
                         t * fout:(t + 1) * fout].set(wpart[t, s * fin:(s + 1) * fin])
        return w

    wagg = bd(postW[:, fin:5 * fin])
    wamp = bd(postW[:, 5 * fin:9 * fin])
    watt = bd(postW[:, 9 * fin:13 * fin])
    wx = jnp.transpose(postW[:, :fin], (1, 0, 2)).reshape(fin, emb)
    postb = p["postb"].reshape(1, emb)
    return dict(wd=wd, ws=ws, wc=wc, cb=cb, pb=pb, wagg=wagg, wamp=wamp,
                watt=watt, wx=wx, postb=postb, lw=p["linW"],
                lb=p["linb"].reshape(1, emb))


def kernel(x, edge_index, edge_attr, batch, params):
    n = x.shape[0]
    e = edge_index.shape[1]
    src = edge_index[0]
    dst = edge_index[1]

    order = jnp.argsort(dst)
    dst_s = dst[order]
    src_s = src[order]
    ea_s = edge_attr[order]

    epad = ((e + 2999) // 3000) * 3000
    if epad != e:
        dst_s = jnp.concatenate([dst_s, jnp.zeros((epad - e,), jnp.int32)])
        src_s = jnp.concatenate([src_s, jnp.zeros((epad - e,), jnp.int32)])
        ea_s = jnp.concatenate([ea_s, jnp.zeros((epad - e, 4), jnp.float32)])

    row_off = jnp.searchsorted(dst_s[:e], jnp.arange(n + 1, dtype=jnp.int32),
                               side="left").astype(jnp.int32)
    deg = (row_off[1:] - row_off[:-1]).astype(jnp.float32).reshape(n, 1)
    targets = (jnp.arange(_NW + 1, dtype=jnp.int32) * e) // _NW
    nsplit = jnp.searchsorted(row_off, targets, side="left").astype(jnp.int32)
    nsplit = jnp.minimum(nsplit, n)
    tes = row_off[nsplit]
    tes = jnp.concatenate([tes, jnp.zeros((48 - (_NW + 1),), jnp.int32)])

    h = x
    for l in range(2):
        p = params["convs"][l]
        fin = h.shape[1]
        k = _T * fin
        kpad = ((k + 127) // 128) * 128
        w = _layer_weights(p, fin, kpad)
        a, b = _compute_ab(h, w["wd"], w["ws"], w["pb"], kpad)
        c = _compute_c(ea_s, w["wc"], w["cb"], kpad)
        su, sq, mnu, mxu = _sc_seg_stats(b, c, src_s, dst_s, tes, n, kpad)
        h = _compute_post(su, sq, mnu, mxu, a, h, deg, w["wagg"], w["wamp"],
                          w["watt"], w["wx"], w["postb"], w["lw"], w["lb"],
                          do_relu=(l == 0))

    w1 = params["W1"]
    b1 = params["b1"].reshape(1, -1)
    w2 = params["W2"]
    b2 = params["b2"].reshape(1, -1)
    w3 = jnp.pad(params["W3"], ((0, 0), (0, 7)))
    b3 = jnp.pad(params["b3"], (0, 7)).reshape(1, 8)
    out = _compute_pool(h, batch.reshape(1, n), w1, b1, w2, b2, w3, b3)
    return out[:, :1]


# speedup vs baseline: 24.5847x; 1.8365x over previous
import functools

import jax
import jax.numpy as jnp
import numpy as np
from jax import lax
from jax.experimental import pallas as pl
from jax.experimental.pallas import tpu as pltpu
from jax.experimental.pallas import tpu_sc as plsc

_T = 6
_NG = 64
_DEG_HIST = np.concatenate([np.zeros(16), np.array([10000.0])])
_BINS = np.arange(len(_DEG_HIST), dtype=np.float64)
_AVG_LOG = float((np.log(_BINS + 1.0) * _DEG_HIST).sum() / _DEG_HIST.sum())

_CH = 24
_NW = 32
_ROW_BLK = 400
_EC_BLK = 1000


def _ab_body(h_ref, wd_ref, ws_ref, pb_ref, a_ref, b_ref):
    h = h_ref[...]
    a_ref[...] = jnp.dot(h, wd_ref[...], preferred_element_type=jnp.float32) + pb_ref[...]
    b_ref[...] = jnp.dot(h, ws_ref[...], preferred_element_type=jnp.float32)


def _compute_ab(h, wd, ws, pb, kpad):
    n = h.shape[0]
    fin = h.shape[1]
    grid = (n // _ROW_BLK,)
    return pl.pallas_call(
        _ab_body,
        grid=grid,
        in_specs=[
            pl.BlockSpec((_ROW_BLK, fin), lambda i: (i, 0)),
            pl.BlockSpec((fin, kpad), lambda i: (0, 0)),
            pl.BlockSpec((fin, kpad), lambda i: (0, 0)),
            pl.BlockSpec((1, kpad), lambda i: (0, 0)),
        ],
        out_specs=[
            pl.BlockSpec((_ROW_BLK, kpad), lambda i: (i, 0)),
            pl.BlockSpec((_ROW_BLK, kpad), lambda i: (i, 0)),
        ],
        out_shape=[
            jax.ShapeDtypeStruct((n, kpad), jnp.float32),
            jax.ShapeDtypeStruct((n, kpad), jnp.float32),
        ],
    )(h, wd, ws, pb)


def _c_body(ea_ref, wc_ref, cb_ref, c_ref):
    ea = ea_ref[...]
    c_ref[...] = jnp.dot(ea, wc_ref[...], preferred_element_type=jnp.float32) + cb_ref[...]


def _compute_c(ea_s, wc, cb, kpad):
    epad = ea_s.shape[0]
    grid = (epad // _EC_BLK,)
    return pl.pallas_call(
        _c_body,
        grid=grid,
        in_specs=[
            pl.BlockSpec((_EC_BLK, 4), lambda i: (i, 0)),
            pl.BlockSpec((4, kpad), lambda i: (0, 0)),
            pl.BlockSpec((1, kpad), lambda i: (0, 0)),
        ],
        out_specs=pl.BlockSpec((_EC_BLK, kpad), lambda i: (i, 0)),
        out_shape=jax.ShapeDtypeStruct((epad, kpad), jnp.float32),
    )(ea_s, wc, cb)


def _sc_seg_stats(b_rows, c_rows, src_s, dst_s, tes, n_nodes, kpad):
    k16 = kpad // 16
    big = jnp.float32(3.0e38)
    mesh = plsc.VectorSubcoreMesh(core_axis_name="c", subcore_axis_name="s",
                                  num_cores=2, num_subcores=16)
    out_sd = jax.ShapeDtypeStruct((n_nodes, kpad), jnp.float32)

    @functools.partial(
        pl.kernel,
        out_type=(out_sd, out_sd, out_sd, out_sd),
        mesh=mesh,
        compiler_params=pltpu.CompilerParams(needs_layout_passes=False),
        scratch_types=[
            pltpu.VMEM((48,), jnp.int32),
            pltpu.VMEM((_CH,), jnp.int32),
            pltpu.VMEM((_CH,), jnp.int32),
            pltpu.VMEM((2, _CH + 16), jnp.int32),
            pltpu.VMEM((2, _CH, kpad), jnp.float32),
            pltpu.VMEM((2, _CH, kpad), jnp.float32),
            pltpu.VMEM((4, kpad), jnp.float32),
            pltpu.SemaphoreType.DMA,
            pltpu.SemaphoreType.DMA,
            pltpu.SemaphoreType.DMA,
        ],
    )
    def kern(b_hbm, c_hbm, src_hbm, dst_hbm, tes_hbm,
             su_hbm, sq_hbm, mn_hbm, mx_hbm,
             tes_v, idx0_v, idx1_v, dst_v, rows_v, c_v, acc_v,
             semf0, semf1, semo):
        wid = lax.axis_index("s") * 2 + lax.axis_index("c")
        pltpu.sync_copy(tes_hbm, tes_v)
        tv = tes_v[pl.ds(wid, 16)]
        e0 = tv[0]
        e1 = tv[1]
        g0 = e0 // _CH
        ng = jnp.maximum((e1 + _CH - 1) // _CH - g0, 0)
        semf = (semf0, semf1)
        idxs = (idx0_v, idx1_v)
        out_refs = (su_hbm, sq_hbm, mn_hbm, mx_hbm)

        def fetch(buf, g):
            pltpu.sync_copy(src_hbm.at[pl.ds(g * _CH, _CH)], idxs[buf])
            pltpu.sync_copy(dst_hbm.at[pl.ds(g * _CH, _CH)],
                            dst_v.at[buf, pl.ds(0, _CH)])
            pltpu.async_copy(b_hbm.at[idxs[buf]], rows_v.at[buf], semf[buf])
            pltpu.async_copy(c_hbm.at[pl.ds(g * _CH, _CH)], c_v.at[buf], semf[buf])

        def wait_fetch(buf):
            pltpu.make_async_copy(b_hbm.at[idxs[buf]], rows_v.at[buf], semf[buf]).wait()
            pltpu.make_async_copy(c_hbm.at[pl.ds(0, _CH)], c_v.at[buf], semf[buf]).wait()

        def flush(node):
            cps = [pltpu.async_copy(acc_v.at[j], out_refs[j].at[node], semo)
                   for j in range(4)]
            for cp in cps:
                cp.wait()

        def init_acc():
            def ibody(k, _):
                ks = pl.ds(k * 16, 16)
                z = jnp.zeros((16,), jnp.float32)
                acc_v[0, ks] = z
                acc_v[1, ks] = z
                acc_v[2, ks] = jnp.full((16,), big, jnp.float32)
                acc_v[3, ks] = jnp.full((16,), -big, jnp.float32)
                return 0
            lax.fori_loop(0, k16, ibody, 0)

        def process_chunk(buf, g, cur):
            lo = jnp.maximum(e0 - g * _CH, 0)
            hi = jnp.minimum(e1 - g * _CH, _CH)
            lane = lax.broadcasted_iota(jnp.int32, (16,), 0)

            def run_body(_, st):
                i, cur = st
                valid = i < hi
                isafe = jnp.minimum(i, _CH - 1)
                w = dst_v[buf, pl.ds(isafe, 16)]
                n = w[0]
                is_new = valid & (n != cur)

                @pl.when(is_new & (cur >= 0))
                def _():
                    flush(cur)

                @pl.when(is_new)
                def _():
                    init_acc()

                cap = jnp.minimum(hi - isafe, 15)
                stop = (w != n) | (lane >= cap)
                rl = jnp.min(jnp.where(stop, lane, 16))
                j = isafe + jnp.maximum(rl, 1)

                @pl.when(valid)
                def _():
                    def kbody(k, _):
                        ks = pl.ds(k * 16, 16)
                        s1 = acc_v[0, ks]
                        s2 = acc_v[1, ks]
                        mn = acc_v[2, ks]
                        mx = acc_v[3, ks]

                        def ebody(e, c4):
                            s1, s2, mn, mx = c4
                            u = rows_v[buf, e, ks] + c_v[buf, e, ks]
                            return (s1 + u, s2 + u * u,
                                    jnp.minimum(mn, u), jnp.maximum(mx, u))

                        s1, s2, mn, mx = lax.fori_loop(isafe, j, ebody,
                                                       (s1, s2, mn, mx))
                        acc_v[0, ks] = s1
                        acc_v[1, ks] = s2
                        acc_v[2, ks] = mn
                        acc_v[3, ks] = mx
                        return 0

                    lax.fori_loop(0, k16, kbody, 0)

                i_next = jnp.where(valid, j, i)
                cur_next = jnp.where(valid, n, cur)
                return (i_next, cur_next)

            _, cur = lax.fori_loop(0, _CH, run_body, (lo, cur))
            return cur

        @pl.when(ng > 0)
        def _():
            fetch(0, g0)

        @pl.when(ng > 1)
        def _():
            fetch(1, g0 + 1)

        def pair_body(p, cur):
            for half in (0, 1):
                gi = 2 * p + half
                g = g0 + gi

                @pl.when(gi < ng)
                def _():
                    wait_fetch(half)

                cur = process_chunk(half, g, cur)

                @pl.when(gi + 2 < ng)
                def _():
                    fetch(half, g + 2)
            return cur

        cur = lax.fori_loop(0, (ng + 1) // 2, pair_body, jnp.int32(-1))

        @pl.when(cur >= 0)
        def _():
            flush(cur)

    return kern(b_rows, c_rows, src_s, dst_s, tes)


def _post_body(su_ref, sq_ref, mn_ref, mx_ref, a_ref, h_ref, deg_ref,
               wagg_ref, wamp_ref, watt_ref, wx_ref, pb_ref, lw_ref, lb_ref,
               out_ref, *, do_relu):
    deg = deg_ref[...]
    a = a_ref[...]
    su = su_ref[...]
    has = deg > 0.0
    cnt = jnp.maximum(deg, 1.0)
    mean = jnp.where(has, (deg * a + su) / cnt, 0.0)
    s2 = deg * a * a + 2.0 * a * su + sq_ref[...]
    var = jnp.where(has, s2 / cnt - mean * mean, 0.0)
    std = jnp.sqrt(jnp.maximum(var, 0.0) + 1e-5)
    mn = jnp.where(has, a + mn_ref[...], 0.0)
    mx = jnp.where(has, a + mx_ref[...], 0.0)
    g = jnp.concatenate([mean, mn, mx, std], axis=1)
    dc = jnp.maximum(deg, 1.0)
    ldc = jnp.log(dc + 1.0)
    sa = ldc * jnp.float32(1.0 / _AVG_LOG)
    st = jnp.float32(_AVG_LOG) / ldc
    p1 = jnp.dot(g, wagg_ref[...], preferred_element_type=jnp.float32)
    p2 = jnp.dot(g, wamp_ref[...], preferred_element_type=jnp.float32)
    p3 = jnp.dot(g, watt_ref[...], preferred_element_type=jnp.float32)
    px = jnp.dot(h_ref[...], wx_ref[...], preferred_element_type=jnp.float32)
    p = px + p1 + sa * p2 + st * p3 + pb_ref[...]
    o = jnp.dot(p, lw_ref[...], preferred_element_type=jnp.float32) + lb_ref[...]
    if do_relu:
        o = jnp.maximum(o, 0.0)
    out_ref[...] = o


def _compute_post(su, sq, mn, mx, a, h, deg, wagg, wamp, watt, wx, pb, lw, lb,
                  do_relu):
    n, kpad = su.shape
    fin = h.shape[1]
    emb = lw.shape[0]
    grid = (n // _ROW_BLK,)
    row = lambda i: (i, 0)
    cst = lambda i: (0, 0)
    return pl.pallas_call(
        functools.partial(_post_body, do_relu=do_relu),
        grid=grid,
        in_specs=[
            pl.BlockSpec((_ROW_BLK, kpad), row),
            pl.BlockSpec((_ROW_BLK, kpad), row),
            pl.BlockSpec((_ROW_BLK, kpad), row),
            pl.BlockSpec((_ROW_BLK, kpad), row),
            pl.BlockSpec((_ROW_BLK, kpad), row),
            pl.BlockSpec((_ROW_BLK, fin), row),
            pl.BlockSpec((_ROW_BLK, 1), row),
            pl.BlockSpec((4 * kpad, emb), cst),
            pl.BlockSpec((4 * kpad, emb), cst),
            pl.BlockSpec((4 * kpad, emb), cst),
            pl.BlockSpec((fin, emb), cst),
            pl.BlockSpec((1, emb), cst),
            pl.BlockSpec((emb, emb), cst),
            pl.BlockSpec((1, emb), cst),
        ],
        out_specs=pl.BlockSpec((_ROW_BLK, emb), row),
        out_shape=jax.ShapeDtypeStruct((n, emb), jnp.float32),
    )(su, sq, mn, mx, a, h, deg, wagg, wamp, watt, wx, pb, lw, lb)


def _pool_body(h_ref, batch_ref, w1_ref, b1_ref, w2_ref, b2_ref,
               w3_ref, b3_ref, out_ref):
    n = h_ref.shape[0]
    bvec = batch_ref[...]
    gid = lax.broadcasted_iota(jnp.int32, (_NG, n), 0)
    oh = jnp.where(gid == bvec, 1.0, 0.0).astype(jnp.float32)
    g = jnp.dot(oh, h_ref[...], preferred_element_type=jnp.float32)
    g = jnp.maximum(jnp.dot(g, w1_ref[...], preferred_element_type=jnp.float32)
                    + b1_ref[...], 0.0)
    g = jnp.maximum(jnp.dot(g, w2_ref[...], preferred_element_type=jnp.float32)
                    + b2_ref[...], 0.0)
    g = jnp.maximum(jnp.dot(g, w3_ref[...], preferred_element_type=jnp.float32)
                    + b3_ref[...], 0.0)
    out_ref[...] = g


def _compute_pool(h, batch, w1, b1, w2, b2, w3, b3):
    n, emb = h.shape
    d1 = w1.shape[1]
    return pl.pallas_call(
        _pool_body,
        in_specs=[
            pl.BlockSpec((n, emb), lambda: (0, 0)),
            pl.BlockSpec((1, n), lambda: (0, 0)),
            pl.BlockSpec((emb, d1), lambda: (0, 0)),
            pl.BlockSpec((1, d1), lambda: (0, 0)),
            pl.BlockSpec((d1, emb), lambda: (0, 0)),
            pl.BlockSpec((1, emb), lambda: (0, 0)),
            pl.BlockSpec((emb, 8), lambda: (0, 0)),
            pl.BlockSpec((1, 8), lambda: (0, 0)),
        ],
        out_specs=pl.BlockSpec((_NG, 8), lambda: (0, 0)),
        out_shape=jax.ShapeDtypeStruct((_NG, 8), jnp.float32),
    )(h, batch, w1, b1, w2, b2, w3, b3)


def _hist_body(dst_ref, hist_ref):
    i = pl.program_id(1)

    @pl.when(i == 0)
    def _():
        hist_ref[...] = jnp.zeros_like(hist_ref)

    node0 = pl.program_id(0) * _HN_BLK
    nodes = node0 + lax.broadcasted_iota(jnp.int32, (1, _HN_BLK), 1)
    dst = dst_ref[...]
    onehot = (dst == nodes).astype(jnp.float32)
    hist_ref[...] += jnp.sum(onehot, axis=0, keepdims=True)


_HE_BLK = 2000
_HN_BLK = 1280


def _compute_deg(dst, n):
    e = dst.shape[0]
    npad = ((n + _HN_BLK - 1) // _HN_BLK) * _HN_BLK
    grid = (npad // _HN_BLK, e // _HE_BLK)
    hist = pl.pallas_call(
        _hist_body,
        grid=grid,
        in_specs=[pl.BlockSpec((_HE_BLK, 1), lambda j, i: (i, 0))],
        out_specs=pl.BlockSpec((1, _HN_BLK), lambda j, i: (0, j)),
        out_shape=jax.ShapeDtypeStruct((1, npad), jnp.float32),
    )(dst.reshape(e, 1))
    return hist.reshape(npad)[:n]


def _layer_weights(p, fin, kpad):
    k = _T * fin
    preW = p["preW"]
    wd = jnp.transpose(preW[:, :fin], (1, 0, 2)).reshape(fin, k)
    ws = jnp.transpose(preW[:, fin:2 * fin], (1, 0, 2)).reshape(fin, k)
    we2 = preW[:, 2 * fin:]
    wc = jnp.einsum("df,tfg->dtg", p["We"], we2).reshape(4, k)
    cb = (jnp.einsum("f,tfg->tg", p["be"], we2)).reshape(1, k)
    pb = p["preb"].reshape(1, k)
    if kpad != k:
        pad = kpad - k
        wd = jnp.pad(wd, ((0, 0), (0, pad)))
        ws = jnp.pad(ws, ((0, 0), (0, pad)))
        wc = jnp.pad(wc, ((0, 0), (0, pad)))
        cb = jnp.pad(cb, ((0, 0), (0, pad)))
        pb = jnp.pad(pb, ((0, 0), (0, pad)))
    postW = p["postW"]
    fout = postW.shape[2]
    emb = _T * fout

    def bd(wpart):
        w = jnp.zeros((4 * kpad, emb), jnp.float32)
        for s in range(4):
            for t in range(_T):
                w = w.at[s * kpad + t * fin:s * kpad + (t + 1) * fin,
                         t * fout:(t + 1) * fout].set(wpart[t, s * fin:(s + 1) * fin])
        return w

    wagg = bd(postW[:, fin:5 * fin])
    wamp = bd(postW[:, 5 * fin:9 * fin])
    watt = bd(postW[:, 9 * fin:13 * fin])
    wx = jnp.transpose(postW[:, :fin], (1, 0, 2)).reshape(fin, emb)
    postb = p["postb"].reshape(1, emb)
    return dict(wd=wd, ws=ws, wc=wc, cb=cb, pb=pb, wagg=wagg, wamp=wamp,
                watt=watt, wx=wx, postb=postb, lw=p["linW"],
                lb=p["linb"].reshape(1, emb))


def kernel(x, edge_index, edge_attr, batch, params):
    n = x.shape[0]
    e = edge_index.shape[1]
    src = edge_index[0]
    dst = edge_index[1]

    order = jnp.argsort(dst)
    dst_s = dst[order]
    src_s = src[order]
    ea_s = edge_attr[order]

    epad = ((e + 2999) // 3000) * 3000
    if epad != e:
        dst_s = jnp.concatenate([dst_s, jnp.zeros((epad - e,), jnp.int32)])
        src_s = jnp.concatenate([src_s, jnp.zeros((epad - e,), jnp.int32)])
        ea_s = jnp.concatenate([ea_s, jnp.zeros((epad - e, 4), jnp.float32)])

    deg_f = _compute_deg(dst, n)
    deg = deg_f.reshape(n, 1)
    row_off = jnp.concatenate([jnp.zeros((1,), jnp.int32),
                               jnp.cumsum(deg_f.astype(jnp.int32))])
    targets = (jnp.arange(_NW + 1, dtype=jnp.int32) * e) // _NW
    nsplit = jnp.sum(row_off[None, :] < targets[:, None], axis=1,
                     dtype=jnp.int32)
    nsplit = jnp.minimum(nsplit, n)
    tes = row_off[nsplit]
    tes = jnp.concatenate([tes, jnp.zeros((48 - (_NW + 1),), jnp.int32)])

    h = x
    for l in range(2):
        p = params["convs"][l]
        fin = h.shape[1]
        k = _T * fin
        kpad = ((k + 127) // 128) * 128
        w = _layer_weights(p, fin, kpad)
        a, b = _compute_ab(h, w["wd"], w["ws"], w["pb"], kpad)
        c = _compute_c(ea_s, w["wc"], w["cb"], kpad)
        su, sq, mnu, mxu = _sc_seg_stats(b, c, src_s, dst_s, tes, n, kpad)
        h = _compute_post(su, sq, mnu, mxu, a, h, deg, w["wagg"], w["wamp"],
                          w["watt"], w["wx"], w["postb"], w["lw"], w["lb"],
                          do_relu=(l == 0))

    w1 = params["W1"]
    b1 = params["b1"].reshape(1, -1)
    w2 = params["W2"]
    b2 = params["b2"].reshape(1, -1)
    w3 = jnp.pad(params["W3"], ((0, 0), (0, 7)))
    b3 = jnp.pad(params["b3"], (0, 7)).reshape(1, 8)
    out = _compute_pool(h, batch.reshape(1, n), w1, b1, w2, b2, w3, b3)
    return out[:, :1]
